# baseline (speedup 1.0000x reference)
import functools

import jax
import jax.numpy as jnp
from jax import lax
from jax.experimental import pallas as pl
from jax.experimental.pallas import tpu as pltpu
from jax.experimental.pallas import tpu_sc as plsc

H = 64
NCORES = 2
NSUB = 16
LANES = 16
CH = 8
UNIT = NSUB * CH * 128


def _mesh():
    return plsc.VectorSubcoreMesh(
        core_axis_name="c", subcore_axis_name="s",
        num_cores=NCORES, num_subcores=NSUB)


def _pad_edges(ei):
    src, dst = ei[0], ei[1]
    e = src.shape[0]
    e_pad = -(-e // UNIT) * UNIT
    pad = e_pad - e
    src = jnp.concatenate([src, jnp.zeros((pad,), jnp.int32)])
    dst = jnp.concatenate([dst, jnp.full((pad,), -1, jnp.int32)])
    return src.reshape(e_pad // 128, 128), dst.reshape(e_pad // 128, 128), e_pad


def _zero_fill(buf, nrows, ngrp):
    z = jnp.zeros((LANES,), jnp.float32)

    def body(j, _):
        buf[j // ngrp, pl.ds((j % ngrp) * LANES, LANES)] = z
        return 0

    lax.fori_loop(0, nrows * ngrp, body, 0)


def _chunks(total, step=128):
    out, off = [], 0
    while off < total:
        n = min(step, total - off)
        out.append((off, n))
        off += n
    return out


@functools.lru_cache(maxsize=None)
def _make_agg(n_types, e_pads, r_half, r_acc):
    zrows = r_acc // NSUB

    def body(*refs):
        ins = refs[:3 * n_types]
        outs = refs[3 * n_types:4 * n_types]
        accs = refs[4 * n_types:5 * n_types]
        (src_stage, dst_stage, src_work, dst_work,
         rows_v, zbuf, sem) = refs[5 * n_types:]
        c = lax.axis_index("c")
        s = lax.axis_index("s")
        base = c * r_half

        _zero_fill(zbuf, 128, 4)
        for t in range(n_types):
            for off, n in _chunks(zrows):
                pltpu.sync_copy(zbuf.at[pl.ds(0, n)],
                                accs[t].at[pl.ds(s * zrows + off, n)])
        plsc.subcore_barrier()

        for t in range(n_types):
            src_hbm, dst_hbm, table = ins[3 * t:3 * t + 3]
            acc = accs[t]
            nch = e_pads[t] // UNIT

            def ch_body(q, _, src_hbm=src_hbm, dst_hbm=dst_hbm,
                        table=table, acc=acc, nch=nch):
                row0 = (s * nch + q) * CH
                pltpu.sync_copy(src_hbm.at[pl.ds(row0, CH)], src_stage)
                pltpu.sync_copy(dst_hbm.at[pl.ds(row0, CH)], dst_stage)

                def blk_body(b, _):
                    def remap(j, _):
                        off = j * LANES
                        d = dst_stage[b, pl.ds(off, LANES)]
                        sv = src_stage[b, pl.ds(off, LANES)]
                        owned = (d >= base) & (d < base + r_half)
                        dst_work[pl.ds(off, LANES)] = jnp.where(
                            owned, d - base, r_half)
                        src_work[pl.ds(off, LANES)] = jnp.where(owned, sv, 0)
                        return 0

                    lax.fori_loop(0, 128 // LANES, remap, 0)
                    pltpu.async_copy(table.at[src_work], rows_v, sem).wait()
                    pltpu.sync_copy(rows_v, acc.at[dst_work], add=True)
                    return 0

                lax.fori_loop(0, CH, blk_body, 0)
                return 0

            lax.fori_loop(0, nch, ch_body, 0)

        plsc.subcore_barrier()
        for t in range(n_types):
            for off, n in _chunks(zrows):
                pltpu.sync_copy(accs[t].at[pl.ds(s * zrows + off, n)],
                                outs[t].at[c, pl.ds(s * zrows + off, n)])

    out_type = [jax.ShapeDtypeStruct((NCORES, r_acc, H), jnp.float32)
                for _ in range(n_types)]
    scratch = ([pltpu.VMEM_SHARED((r_acc, H), jnp.float32)
                for _ in range(n_types)] +
               [pltpu.VMEM((CH, 128), jnp.int32),
                pltpu.VMEM((CH, 128), jnp.int32),
                pltpu.VMEM((128,), jnp.int32),
                pltpu.VMEM((128,), jnp.int32),
                pltpu.VMEM((128, H), jnp.float32),
                pltpu.VMEM((128, H), jnp.float32),
                pltpu.SemaphoreType.DMA])
    return pl.kernel(body, out_type=out_type, mesh=_mesh(),
                     scratch_types=scratch,
                     compiler_params=pltpu.CompilerParams(
                         use_tc_tiling_on_sc=False))


@functools.lru_cache(maxsize=None)
def _make_counts(county_e_pads, case_e_pads, r_halves, r_accs):
    groups = (county_e_pads, case_e_pads)
    n_in = len(county_e_pads) + len(case_e_pads)

    def body(*refs):
        ins = refs[:n_in]
        outs = refs[n_in:n_in + 2]
        accs = refs[n_in + 2:n_in + 4]
        dst_stage, dst_work, ones_v, zbuf = refs[n_in + 4:]
        c = lax.axis_index("c")
        s = lax.axis_index("s")

        _zero_fill(zbuf, 128, 1)
        for g in range(2):
            zrows = r_accs[g] // NSUB
            for off, n in _chunks(zrows):
                pltpu.sync_copy(zbuf.at[pl.ds(0, n)],
                                accs[g].at[pl.ds(s * zrows + off, n)])
        one = jnp.ones((LANES,), jnp.float32)

        def fill_ones(j, _):
            ones_v[j, pl.ds(0, LANES)] = one
            return 0

        lax.fori_loop(0, 128, fill_ones, 0)
        plsc.subcore_barrier()

        k = 0
        for g in range(2):
            base = c * r_halves[g]
            for e_pad in groups[g]:
                dst_hbm = ins[k]
                k += 1
                acc = accs[g]
                nch = e_pad // UNIT

                def ch_body(q, _, dst_hbm=dst_hbm, acc=acc, nch=nch,
                            base=base, r_half=r_halves[g]):
                    row0 = (s * nch + q) * CH
                    pltpu.sync_copy(dst_hbm.at[pl.ds(row0, CH)], dst_stage)

                    def blk_body(b, _):
                        def remap(j, _):
                            off = j * LANES
                            d = dst_stage[b, pl.ds(off, LANES)]
                            owned = (d >= base) & (d < base + r_half)
                            dst_work[pl.ds(off, LANES)] = jnp.where(
                                owned, d - base, r_half)
                            return 0

                        lax.fori_loop(0, 128 // LANES, remap, 0)
                        pltpu.sync_copy(ones_v, acc.at[dst_work], add=True)
                        return 0

                    lax.fori_loop(0, CH, blk_body, 0)
                    return 0

                lax.fori_loop(0, nch, ch_body, 0)

        plsc.subcore_barrier()
        for g in range(2):
            zrows = r_accs[g] // NSUB
            for off, n in _chunks(zrows):
                pltpu.sync_copy(accs[g].at[pl.ds(s * zrows + off, n)],
                                outs[g].at[c, pl.ds(s * zrows + off, n)])

    out_type = [jax.ShapeDtypeStruct((NCORES, r_accs[0], LANES), jnp.float32),
                jax.ShapeDtypeStruct((NCORES, r_accs[1], LANES), jnp.float32)]
    scratch = [pltpu.VMEM_SHARED((r_accs[0], LANES), jnp.float32),
               pltpu.VMEM_SHARED((r_accs[1], LANES), jnp.float32),
               pltpu.VMEM((CH, 128), jnp.int32),
               pltpu.VMEM((128,), jnp.int32),
               pltpu.VMEM((128, LANES), jnp.float32),
               pltpu.VMEM((128, LANES), jnp.float32)]
    return pl.kernel(body, out_type=out_type, mesh=_mesh(),
                     scratch_types=scratch,
                     compiler_params=pltpu.CompilerParams(
                         use_tc_tiling_on_sc=False))


def _enc_body(x_ref, w_ref, b_ref, o_ref):
    o_ref[...] = jnp.dot(x_ref[...], w_ref[...],
                         preferred_element_type=jnp.float32) + b_ref[...]


def _encode(x, w, b, blk):
    n, f = x.shape
    return pl.pallas_call(
        _enc_body,
        grid=(n // blk,),
        in_specs=[pl.BlockSpec((blk, f), lambda i: (i, 0)),
                  pl.BlockSpec((f, H), lambda i: (0, 0)),
                  pl.BlockSpec((1, H), lambda i: (0, 0))],
        out_specs=pl.BlockSpec((blk, H), lambda i: (i, 0)),
        out_shape=jax.ShapeDtypeStruct((n, H), jnp.float32),
    )(x, w, b.reshape(1, H))


def _upd2_body(h_ref, a1_ref, a2_ref, cnt_ref, w1_ref, w2_ref, b_ref, o_ref):
    m = (jnp.dot(a1_ref[0], w1_ref[...], preferred_element_type=jnp.float32)
         + jnp.dot(a2_ref[0], w2_ref[...], preferred_element_type=jnp.float32))
    m = m / jnp.maximum(cnt_ref[0][:, 0:1], 1.0)
    o_ref[...] = h_ref[...] + jnp.maximum(m + b_ref[...], 0.0)


def _upd1_body(h_ref, a1_ref, cnt_ref, w1_ref, b_ref, o_ref):
    m = jnp.dot(a1_ref[0], w1_ref[...], preferred_element_type=jnp.float32)
    m = m / jnp.maximum(cnt_ref[0][:, 0:1], 1.0)
    o_ref[...] = h_ref[...] + jnp.maximum(m + b_ref[...], 0.0)


def _update(h, aggs, cnt_parts, ws, b, r_half, blk):
    n = h.shape[0]
    gc = r_half // blk
    agg_spec = pl.BlockSpec((1, blk, H), lambda i: (i // gc, i % gc, 0))
    cnt_spec = pl.BlockSpec((1, blk, LANES), lambda i: (i // gc, i % gc, 0))
    w_spec = pl.BlockSpec((H, H), lambda i: (0, 0))
    body = _upd2_body if len(aggs) == 2 else _upd1_body
    in_specs = ([pl.BlockSpec((blk, H), lambda i: (i, 0))]
                + [agg_spec] * len(aggs) + [cnt_spec]
                + [w_spec] * len(ws)
                + [pl.BlockSpec((1, H), lambda i: (0, 0))])
    return pl.pallas_call(
        body,
        grid=(n // blk,),
        in_specs=in_specs,
        out_specs=pl.BlockSpec((blk, H), lambda i: (i, 0)),
        out_shape=jax.ShapeDtypeStruct((n, H), jnp.float32),
    )(h, *aggs, cnt_parts, *ws, b.reshape(1, H))


NC = 10000
NCASE = 50000
RC = NC // NCORES
RS = NCASE // NCORES
RC_ACC = -(-(RC + 1) // 128) * 128
RS_ACC = -(-(RS + 1) // 128) * 128
NITER = 3


def kernel(x_county, x_case, ei_spatial, ei_genetic, ei_belongs,
           W_enc_county, b_enc_county, W_enc_case, b_enc_case,
           W_spatial, W_genetic, W_belongs, b_county, b_case):
    src_sp, dst_sp, ep_sp = _pad_edges(ei_spatial)
    src_ge, dst_ge, ep_ge = _pad_edges(ei_genetic)
    src_bl, dst_bl, ep_bl = _pad_edges(ei_belongs)

    hc = _encode(x_county, W_enc_county, b_enc_county, 2000)
    hs = _encode(x_case, W_enc_case, b_enc_case, 2000)

    cnt_c, cnt_s = _make_counts((ep_sp, ep_bl), (ep_ge,),
                                (RC, RS), (RC_ACC, RS_ACC))(
        dst_sp, dst_bl, dst_ge)

    county_agg = _make_agg(2, (ep_sp, ep_bl), RC, RC_ACC)
    case_agg = _make_agg(1, (ep_ge,), RS, RS_ACC)

    for _ in range(NITER):
        agg_sp, agg_bl = county_agg(src_sp, dst_sp, hc, src_bl, dst_bl, hs)
        (agg_ge,) = case_agg(src_ge, dst_ge, hs)
        hc = _update(hc, [agg_sp, agg_bl], cnt_c,
                     [W_spatial, W_belongs], b_county, RC, 1000)
        hs = _update(hs, [agg_ge], cnt_s,
                     [W_genetic], b_case, RS, 1000)
    return hc, hs


# speedup vs baseline: 1.0475x; 1.0475x over previous
import functools

import jax
import jax.numpy as jnp
from jax import lax
from jax.experimental import pallas as pl
from jax.experimental.pallas import tpu as pltpu
from jax.experimental.pallas import tpu_sc as plsc

H = 64
NCORES = 2
NSUB = 16
LANES = 16
NBUF = 12
GG = 4
BLK_C = 64
BLK_G = 32


def _mesh():
    return plsc.VectorSubcoreMesh(
        core_axis_name="c", subcore_axis_name="s",
        num_cores=NCORES, num_subcores=NSUB)


def _pad_edges(ei, blk):
    src, dst = ei[0], ei[1]
    e = src.shape[0]
    unit = NSUB * blk
    e_pad = -(-e // unit) * unit
    pad = e_pad - e
    src = jnp.concatenate([src, jnp.zeros((pad,), jnp.int32)])
    dst = jnp.concatenate([dst, jnp.full((pad,), -1, jnp.int32)])
    return src.reshape(e_pad // blk, blk), dst.reshape(e_pad // blk, blk), e_pad


def _zero_fill(buf, nrows, ngrp):
    z = jnp.zeros((LANES,), jnp.float32)

    def body(j, _):
        buf[j // ngrp, pl.ds((j % ngrp) * LANES, LANES)] = z
        return 0

    lax.fori_loop(0, nrows * ngrp, body, 0)


def _chunks(total, step):
    out, off = [], 0
    while off < total:
        n = min(step, total - off)
        out.append((off, n))
        off += n
    return out


def _remap_block(src_stage, dst_stage, i, src_work, dst_work,
                 base, r_half, blk):
    lane = lax.iota(jnp.int32, LANES)
    for j in range(blk // LANES):
        off = j * LANES
        dummy = r_half + ((i * (blk // LANES) + j) % 4) * LANES + lane
        d = dst_stage[i, pl.ds(off, LANES)]
        owned = (d >= base) & (d < base + r_half)
        dst_work[pl.ds(off, LANES)] = jnp.where(owned, d - base, dummy)
        if src_stage is not None:
            sv = src_stage[i, pl.ds(off, LANES)]
            src_work[pl.ds(off, LANES)] = jnp.where(owned, sv, 0)


def _acc_zero_and_barrier(accs, s):
    for acc, zbuf in accs:
        zrows = acc.shape[0] // NSUB
        znr = zbuf.shape[0]
        for off, n in _chunks(zrows, znr):
            pltpu.sync_copy(zbuf.at[pl.ds(0, n)],
                            acc.at[pl.ds(s * zrows + off, n)])
    plsc.subcore_barrier()


def _acc_write_out(accs_outs, s, c):
    plsc.subcore_barrier()
    for acc, out in accs_outs:
        zrows = acc.shape[0] // NSUB
        for off, n in _chunks(zrows, 256):
            pltpu.sync_copy(acc.at[pl.ds(s * zrows + off, n)],
                            out.at[c, pl.ds(s * zrows + off, n)])


@functools.lru_cache(maxsize=None)
def _make_agg(specs, r_half, r_acc):
    n_types = len(specs)
    blks = sorted({blk for _, blk in specs})

    def body(*refs):
        ins = refs[:3 * n_types]
        outs = refs[3 * n_types:4 * n_types]
        accs = refs[4 * n_types:5 * n_types]
        rest = list(refs[5 * n_types:])
        stages = {}
        works = {}
        rows = {}
        for blk in blks:
            stages[blk] = (rest.pop(0), rest.pop(0))
            works[blk] = (rest[:NBUF], rest[NBUF:2 * NBUF])
            del rest[:2 * NBUF]
            rows[blk] = rest[:NBUF]
            del rest[:NBUF]
        sems = rest[:NBUF]
        c = lax.axis_index("c")
        s = lax.axis_index("s")
        base = c * r_half

        zbuf = rows[blks[-1]][0]
        _zero_fill(zbuf, zbuf.shape[0], 4)
        _acc_zero_and_barrier([(acc, zbuf) for acc in accs], s)

        for t, (e_pad, blk) in enumerate(specs):
            src_hbm, dst_hbm, table = ins[3 * t:3 * t + 3]
            acc = accs[t]
            src_stage, dst_stage = stages[blk]
            src_work, dst_work = works[blk]
            row_bufs = rows[blk]
            nblk = e_pad // (NSUB * blk)
            sg = GG * NBUF
            nsg, tail = divmod(nblk, sg)

            def do_blocks(i0, nb, src_stage=src_stage, dst_stage=dst_stage,
                          table=table, acc=acc, src_work=src_work,
                          dst_work=dst_work, row_bufs=row_bufs, blk=blk):
                gh = []
                for i in range(nb):
                    _remap_block(src_stage, dst_stage, i0 + i,
                                 src_work[i], dst_work[i], base, r_half, blk)
                    gh.append(pltpu.async_copy(
                        table.at[src_work[i]], row_bufs[i], sems[i]))
                sh = []
                for i in range(nb):
                    gh[i].wait()
                    sh.append(pltpu.async_copy(
                        row_bufs[i], acc.at[dst_work[i]], sems[i], add=True))
                for i in range(nb):
                    sh[i].wait()

            def sg_body(q, _, src_hbm=src_hbm, dst_hbm=dst_hbm,
                        src_stage=src_stage, dst_stage=dst_stage,
                        do_blocks=do_blocks, nblk=nblk, sg=sg):
                row0 = s * nblk + q * sg
                pltpu.sync_copy(src_hbm.at[pl.ds(row0, sg)], src_stage)
                pltpu.sync_copy(dst_hbm.at[pl.ds(row0, sg)], dst_stage)
                for gg in range(GG):
                    do_blocks(gg * NBUF, NBUF)
                return 0

            lax.fori_loop(0, nsg, sg_body, 0)
            if tail:
                row0 = s * nblk + nsg * sg
                pltpu.sync_copy(src_hbm.at[pl.ds(row0, tail)],
                                src_stage.at[pl.ds(0, tail)])
                pltpu.sync_copy(dst_hbm.at[pl.ds(row0, tail)],
                                dst_stage.at[pl.ds(0, tail)])
                for i0, nb in _chunks(tail, NBUF):
                    do_blocks(i0, nb)

        _acc_write_out(list(zip(accs, outs)), s, c)

    out_type = [jax.ShapeDtypeStruct((NCORES, r_acc, H), jnp.float32)
                for _ in range(n_types)]
    scratch = [pltpu.VMEM_SHARED((r_acc, H), jnp.float32)
               for _ in range(n_types)]
    for blk in blks:
        scratch += [pltpu.VMEM((GG * NBUF, blk), jnp.int32),
                    pltpu.VMEM((GG * NBUF, blk), jnp.int32)]
        scratch += [pltpu.VMEM((blk,), jnp.int32) for _ in range(2 * NBUF)]
        scratch += [pltpu.VMEM((blk, H), jnp.float32) for _ in range(NBUF)]
    scratch += [pltpu.SemaphoreType.DMA for _ in range(NBUF)]
    return pl.kernel(body, out_type=out_type, mesh=_mesh(),
                     scratch_types=scratch,
                     compiler_params=pltpu.CompilerParams(
                         use_tc_tiling_on_sc=False))


@functools.lru_cache(maxsize=None)
def _make_counts(county_specs, case_specs, r_halves, r_accs):
    groups = (county_specs, case_specs)
    n_in = len(county_specs) + len(case_specs)
    blks = sorted({blk for g in groups for _, blk in g})

    def body(*refs):
        ins = refs[:n_in]
        outs = refs[n_in:n_in + 2]
        accs = refs[n_in + 2:n_in + 4]
        rest = list(refs[n_in + 4:])
        stages = {}
        works = {}
        ones = {}
        for blk in blks:
            stages[blk] = rest.pop(0)
            works[blk] = rest[:NBUF]
            del rest[:NBUF]
            ones[blk] = rest.pop(0)
        zbuf = rest.pop(0)
        sems = rest[:NBUF]
        c = lax.axis_index("c")
        s = lax.axis_index("s")

        _zero_fill(zbuf, 128, 1)
        _acc_zero_and_barrier([(acc, zbuf) for acc in accs], s)
        one = jnp.ones((LANES,), jnp.float32)
        for blk in blks:

            def fill_ones(j, _, ov=ones[blk]):
                ov[j, pl.ds(0, LANES)] = one
                return 0

            lax.fori_loop(0, blk, fill_ones, 0)

        k = 0
        for g in range(2):
            base = c * r_halves[g]
            r_half = r_halves[g]
            for e_pad, blk in groups[g]:
                dst_hbm = ins[k]
                k += 1
                acc = accs[g]
                dst_stage = stages[blk]
                dst_work = works[blk]
                ones_v = ones[blk]
                nblk = e_pad // (NSUB * blk)
                sg = GG * NBUF
                nsg, tail = divmod(nblk, sg)

                def do_blocks(i0, nb, dst_stage=dst_stage, acc=acc,
                              dst_work=dst_work, ones_v=ones_v,
                              base=base, r_half=r_half, blk=blk):
                    for i in range(nb):
                        _remap_block(None, dst_stage, i0 + i,
                                     None, dst_work[i], base, r_half, blk)
                    sh = []
                    for i in range(nb):
                        sh.append(pltpu.async_copy(
                            ones_v, acc.at[dst_work[i]], sems[i], add=True))
                    for i in range(nb):
                        sh[i].wait()

                def sg_body(q, _, dst_hbm=dst_hbm, dst_stage=dst_stage,
                            do_blocks=do_blocks, nblk=nblk, sg=sg):
                    row0 = s * nblk + q * sg
                    pltpu.sync_copy(dst_hbm.at[pl.ds(row0, sg)], dst_stage)
                    for gg in range(GG):
                        do_blocks(gg * NBUF, NBUF)
                    return 0

                lax.fori_loop(0, nsg, sg_body, 0)
                if tail:
                    row0 = s * nblk + nsg * sg
                    pltpu.sync_copy(dst_hbm.at[pl.ds(row0, tail)],
                                    dst_stage.at[pl.ds(0, tail)])
                    for i0, nb in _chunks(tail, NBUF):
                        do_blocks(i0, nb)

        _acc_write_out(list(zip(accs, outs)), s, c)

    out_type = [jax.ShapeDtypeStruct((NCORES, r_accs[0], LANES), jnp.float32),
                jax.ShapeDtypeStruct((NCORES, r_accs[1], LANES), jnp.float32)]
    scratch = [pltpu.VMEM_SHARED((r_accs[0], LANES), jnp.float32),
               pltpu.VMEM_SHARED((r_accs[1], LANES), jnp.float32)]
    for blk in blks:
        scratch += [pltpu.VMEM((GG * NBUF, blk), jnp.int32)]
        scratch += [pltpu.VMEM((blk,), jnp.int32) for _ in range(NBUF)]
        scratch += [pltpu.VMEM((blk, LANES), jnp.float32)]
    scratch += [pltpu.VMEM((128, LANES), jnp.float32)]
    scratch += [pltpu.SemaphoreType.DMA for _ in range(NBUF)]
    return pl.kernel(body, out_type=out_type, mesh=_mesh(),
                     scratch_types=scratch,
                     compiler_params=pltpu.CompilerParams(
                         use_tc_tiling_on_sc=False))


def _enc_body(x_ref, w_ref, b_ref, o_ref):
    o_ref[...] = jnp.dot(x_ref[...], w_ref[...],
                         preferred_element_type=jnp.float32) + b_ref[...]


def _encode(x, w, b, blk):
    n, f = x.shape
    return pl.pallas_call(
        _enc_body,
        grid=(n // blk,),
        in_specs=[pl.BlockSpec((blk, f), lambda i: (i, 0)),
                  pl.BlockSpec((f, H), lambda i: (0, 0)),
                  pl.BlockSpec((1, H), lambda i: (0, 0))],
        out_specs=pl.BlockSpec((blk, H), lambda i: (i, 0)),
        out_shape=jax.ShapeDtypeStruct((n, H), jnp.float32),
    )(x, w, b.reshape(1, H))


def _upd2_body(h_ref, a1_ref, a2_ref, cnt_ref, w1_ref, w2_ref, b_ref, o_ref):
    m = (jnp.dot(a1_ref[0], w1_ref[...], preferred_element_type=jnp.float32)
         + jnp.dot(a2_ref[0], w2_ref[...], preferred_element_type=jnp.float32))
    m = m / jnp.maximum(cnt_ref[0][:, 0:1], 1.0)
    o_ref[...] = h_ref[...] + jnp.maximum(m + b_ref[...], 0.0)


def _upd1_body(h_ref, a1_ref, cnt_ref, w1_ref, b_ref, o_ref):
    m = jnp.dot(a1_ref[0], w1_ref[...], preferred_element_type=jnp.float32)
    m = m / jnp.maximum(cnt_ref[0][:, 0:1], 1.0)
    o_ref[...] = h_ref[...] + jnp.maximum(m + b_ref[...], 0.0)


def _update(h, aggs, cnt_parts, ws, b, r_half, blk):
    n = h.shape[0]
    gc = r_half // blk
    agg_spec = pl.BlockSpec((1, blk, H), lambda i: (i // gc, i % gc, 0))
    cnt_spec = pl.BlockSpec((1, blk, LANES), lambda i: (i // gc, i % gc, 0))
    w_spec = pl.BlockSpec((H, H), lambda i: (0, 0))
    body = _upd2_body if len(aggs) == 2 else _upd1_body
    in_specs = ([pl.BlockSpec((blk, H), lambda i: (i, 0))]
                + [agg_spec] * len(aggs) + [cnt_spec]
                + [w_spec] * len(ws)
                + [pl.BlockSpec((1, H), lambda i: (0, 0))])
    return pl.pallas_call(
        body,
        grid=(n // blk,),
        in_specs=in_specs,
        out_specs=pl.BlockSpec((blk, H), lambda i: (i, 0)),
        out_shape=jax.ShapeDtypeStruct((n, H), jnp.float32),
    )(h, *aggs, cnt_parts, *ws, b.reshape(1, H))


NC = 10000
NCASE = 50000
RC = NC // NCORES
RS = NCASE // NCORES
RC_ACC = -(-(RC + 64) // 128) * 128
RS_ACC = -(-(RS + 64) // 128) * 128
NITER = 3


def kernel(x_county, x_case, ei_spatial, ei_genetic, ei_belongs,
           W_enc_county, b_enc_county, W_enc_case, b_enc_case,
           W_spatial, W_genetic, W_belongs, b_county, b_case):
    src_sp, dst_sp, ep_sp = _pad_edges(ei_spatial, BLK_C)
    src_ge, dst_ge, ep_ge = _pad_edges(ei_genetic, BLK_G)
    src_bl, dst_bl, ep_bl = _pad_edges(ei_belongs, BLK_C)

    hc = _encode(x_county, W_enc_county, b_enc_county, 2000)
    hs = _encode(x_case, W_enc_case, b_enc_case, 2000)

    cnt_c, cnt_s = _make_counts(((ep_sp, BLK_C), (ep_bl, BLK_C)),
                                ((ep_ge, BLK_G),),
                                (RC, RS), (RC_ACC, RS_ACC))(
        dst_sp, dst_bl, dst_ge)

    county_agg = _make_agg(((ep_sp, BLK_C), (ep_bl, BLK_C)), RC, RC_ACC)
    case_agg = _make_agg(((ep_ge, BLK_G),), RS, RS_ACC)

    for _ in range(NITER):
        agg_sp, agg_bl = county_agg(src_sp, dst_sp, hc, src_bl, dst_bl, hs)
        (agg_ge,) = case_agg(src_ge, dst_ge, hs)
        hc = _update(hc, [agg_sp, agg_bl], cnt_c,
                     [W_spatial, W_belongs], b_county, RC, 1000)
        hs = _update(hs, [agg_ge], cnt_s,
                     [W_genetic], b_case, RS, 1000)
    return hc, hs
